# Initial kernel scaffold; baseline (speedup 1.0000x reference)
import jax
import jax.numpy as jnp
from jax.experimental import pallas as pl


def kernel(x, edge_index, Wg, bg, Wc, bc):
    raise NotImplementedError("write your pallas kernel here")


# speedup vs baseline: 19.0582x; 19.0582x over previous
import functools

import jax
import jax.numpy as jnp
from jax import lax
from jax.experimental import pallas as pl
from jax.experimental.pallas import tpu as pltpu
from jax.experimental.pallas import tpu_sc as plsc

N = 100000
E = 1600000
D = 128
GDIM = 7
HDIM = 256

BN = 2048
GRID = 49
NP = GRID * BN
NSUB = 16
SLICE = NP // NSUB

CH = 128
ROWS = 13
OUTER = 30
NCHUNK = E // CH
BASE_CHUNKS = 390
EXTRA_TILES = NCHUNK - 32 * BASE_CHUNKS

@functools.lru_cache(maxsize=None)
def _get_mesh():
    return plsc.VectorSubcoreMesh(core_axis_name="c", subcore_axis_name="s")


def _zero_slice(zero_v, shared, s):
    pltpu.sync_copy(zero_v, shared.at[pl.ds(s * SLICE, SLICE)])


def _tile_chunk_start(w):
    return jnp.where(w < EXTRA_TILES,
                     w * (BASE_CHUNKS + 1),
                     EXTRA_TILES * (BASE_CHUNKS + 1) + (w - EXTRA_TILES) * BASE_CHUNKS)


def _sc_hist(edge_hbm, outd_a, outd_b, ind_a, ind_b,
             src_v, dst_v, ones_v, zero_v, acc_out, acc_in):
    c = lax.axis_index("c")
    s = lax.axis_index("s")
    w = c * NSUB + s

    def fill_ones(i, _):
        ones_v[pl.ds(i * 16, 16)] = jnp.ones((16,), jnp.float32)
        return _
    lax.fori_loop(0, CH // 16, fill_ones, None)

    def fill_zero(i, _):
        zero_v[pl.ds(i * 16, 16)] = jnp.zeros((16,), jnp.float32)
        return _
    lax.fori_loop(0, SLICE // 16, fill_zero, None)

    _zero_slice(zero_v, acc_out, s)
    _zero_slice(zero_v, acc_in, s)
    plsc.subcore_barrier()

    start = _tile_chunk_start(w)

    def outer(o, _):
        row0 = start + o * ROWS
        pltpu.sync_copy(edge_hbm.at[0, pl.ds(row0, ROWS)], src_v)
        pltpu.sync_copy(edge_hbm.at[1, pl.ds(row0, ROWS)], dst_v)
        for j in range(ROWS):
            pltpu.sync_copy(ones_v, acc_out.at[src_v.at[j]], add=True)
            pltpu.sync_copy(ones_v, acc_in.at[dst_v.at[j]], add=True)
        return _
    lax.fori_loop(0, OUTER, outer, None)

    @pl.when(w < EXTRA_TILES)
    def _extra():
        row = start + BASE_CHUNKS
        pltpu.sync_copy(edge_hbm.at[0, pl.ds(row, 1)], src_v.at[pl.ds(0, 1)])
        pltpu.sync_copy(edge_hbm.at[1, pl.ds(row, 1)], dst_v.at[pl.ds(0, 1)])
        pltpu.sync_copy(ones_v, acc_out.at[src_v.at[0]], add=True)
        pltpu.sync_copy(ones_v, acc_in.at[dst_v.at[0]], add=True)

    plsc.subcore_barrier()
    off = s * SLICE

    @pl.when(c == 0)
    def _w0():
        pltpu.sync_copy(acc_out.at[pl.ds(off, SLICE)], outd_a.at[pl.ds(off, SLICE)])
        pltpu.sync_copy(acc_in.at[pl.ds(off, SLICE)], ind_a.at[pl.ds(off, SLICE)])

    @pl.when(c == 1)
    def _w1():
        pltpu.sync_copy(acc_out.at[pl.ds(off, SLICE)], outd_b.at[pl.ds(off, SLICE)])
        pltpu.sync_copy(acc_in.at[pl.ds(off, SLICE)], ind_b.at[pl.ds(off, SLICE)])


def _sc_edge(edge_hbm, q_hbm, t_a, t_b,
             src_v, dst_v, val_v, zero_v, acc_t):
    c = lax.axis_index("c")
    s = lax.axis_index("s")
    w = c * NSUB + s

    def fill_zero(i, _):
        zero_v[pl.ds(i * 16, 16)] = jnp.zeros((16,), jnp.float32)
        return _
    lax.fori_loop(0, SLICE // 16, fill_zero, None)

    _zero_slice(zero_v, acc_t, s)
    plsc.subcore_barrier()

    start = _tile_chunk_start(w)

    def outer(o, _):
        row0 = start + o * ROWS
        pltpu.sync_copy(edge_hbm.at[0, pl.ds(row0, ROWS)], src_v)
        pltpu.sync_copy(edge_hbm.at[1, pl.ds(row0, ROWS)], dst_v)
        for j in range(ROWS):
            pltpu.sync_copy(q_hbm.at[dst_v.at[j]], val_v.at[j])
        for j in range(ROWS):
            pltpu.sync_copy(val_v.at[j], acc_t.at[src_v.at[j]], add=True)
        return _
    lax.fori_loop(0, OUTER, outer, None)

    @pl.when(w < EXTRA_TILES)
    def _extra():
        row = start + BASE_CHUNKS
        pltpu.sync_copy(edge_hbm.at[0, pl.ds(row, 1)], src_v.at[pl.ds(0, 1)])
        pltpu.sync_copy(edge_hbm.at[1, pl.ds(row, 1)], dst_v.at[pl.ds(0, 1)])
        pltpu.sync_copy(q_hbm.at[dst_v.at[0]], val_v.at[0])
        pltpu.sync_copy(val_v.at[0], acc_t.at[src_v.at[0]], add=True)

    plsc.subcore_barrier()
    off = s * SLICE

    @pl.when(c == 0)
    def _w0():
        pltpu.sync_copy(acc_t.at[pl.ds(off, SLICE)], t_a.at[pl.ds(off, SLICE)])

    @pl.when(c == 1)
    def _w1():
        pltpu.sync_copy(acc_t.at[pl.ds(off, SLICE)], t_b.at[pl.ds(off, SLICE)])


def _tc_gate_body(x_ref, wg_ref, bg_ref, gt_ref):
    z = lax.dot_general(wg_ref[...], x_ref[...],
                        (((1,), (1,)), ((), ())),
                        preferred_element_type=jnp.float32)
    z = z + bg_ref[:, 0:1]
    gt_ref[...] = 1.0 / (1.0 + jnp.exp(-z))


def _tc_q_body(ia_ref, ib_ref, q_ref):
    d = jnp.maximum(ia_ref[...] + ib_ref[...], 1.0)
    q_ref[...] = lax.rsqrt(d)


def _tc_final_body(gt_ref, oa_ref, ob_ref, ta_ref, tb_ref, wc_ref, bc_ref,
                   out_ref, acc_ref):
    j = pl.program_id(0)

    @pl.when(j == 0)
    def _init():
        acc_ref[...] = jnp.zeros((8, 128), jnp.float32)

    outd = jnp.maximum(oa_ref[...] + ob_ref[...], 1.0)
    cvec = lax.rsqrt(outd) * (ta_ref[...] + tb_ref[...])
    node = j * BN + lax.broadcasted_iota(jnp.int32, (1, BN), 1)
    prod = jnp.where(node < N, gt_ref[...] * cvec, 0.0)
    acc = acc_ref[...]
    for k in range(BN // 128):
        acc = acc + prod[:, k * 128:(k + 1) * 128]
    acc_ref[...] = acc

    @pl.when(j == GRID - 1)
    def _fin():
        s8 = jnp.sum(acc_ref[...], axis=1, keepdims=True)
        out = lax.dot_general(s8, wc_ref[...],
                              (((0,), (0,)), ((), ())),
                              preferred_element_type=jnp.float32)
        out_ref[...] = out + float(N) * bc_ref[...]


@functools.lru_cache(maxsize=None)
def _get_sc_kernels():
    mesh = _get_mesh()
    params = pltpu.CompilerParams(use_tc_tiling_on_sc=False)
    hist = functools.partial(
        pl.kernel,
        mesh=mesh,
        compiler_params=params,
        out_type=[jax.ShapeDtypeStruct((NP,), jnp.float32) for _ in range(4)],
        scratch_types=[
            pltpu.VMEM((ROWS, CH), jnp.int32),
            pltpu.VMEM((ROWS, CH), jnp.int32),
            pltpu.VMEM((CH,), jnp.float32),
            pltpu.VMEM((SLICE,), jnp.float32),
            pltpu.VMEM_SHARED((NP,), jnp.float32),
            pltpu.VMEM_SHARED((NP,), jnp.float32),
        ],
    )(_sc_hist)
    edge = functools.partial(
        pl.kernel,
        mesh=mesh,
        compiler_params=params,
        out_type=[jax.ShapeDtypeStruct((NP,), jnp.float32) for _ in range(2)],
        scratch_types=[
            pltpu.VMEM((ROWS, CH), jnp.int32),
            pltpu.VMEM((ROWS, CH), jnp.int32),
            pltpu.VMEM((ROWS, CH), jnp.float32),
            pltpu.VMEM((SLICE,), jnp.float32),
            pltpu.VMEM_SHARED((NP,), jnp.float32),
        ],
    )(_sc_edge)
    return hist, edge


def kernel(x, edge_index, Wg, bg, Wc, bc):
    edge3 = edge_index.reshape(2, NCHUNK, CH)
    sc_hist, sc_edge = _get_sc_kernels()

    outd_a, outd_b, ind_a, ind_b = sc_hist(edge3)

    q = pl.pallas_call(
        _tc_q_body,
        grid=(GRID,),
        in_specs=[pl.BlockSpec((1, BN), lambda j: (0, j)),
                  pl.BlockSpec((1, BN), lambda j: (0, j))],
        out_specs=pl.BlockSpec((1, BN), lambda j: (0, j)),
        out_shape=jax.ShapeDtypeStruct((1, NP), jnp.float32),
    )(ind_a.reshape(1, NP), ind_b.reshape(1, NP))

    t_a, t_b = sc_edge(edge3, q.reshape(NP))

    Wg8 = jnp.zeros((8, D), jnp.float32).at[:GDIM].set(Wg)
    bg8 = jnp.broadcast_to(
        jnp.zeros((8,), jnp.float32).at[:GDIM].set(bg)[:, None], (8, 128))
    gt = pl.pallas_call(
        _tc_gate_body,
        grid=(GRID,),
        in_specs=[pl.BlockSpec((BN, D), lambda j: (j, 0)),
                  pl.BlockSpec((8, D), lambda j: (0, 0)),
                  pl.BlockSpec((8, 128), lambda j: (0, 0))],
        out_specs=pl.BlockSpec((8, BN), lambda j: (0, j)),
        out_shape=jax.ShapeDtypeStruct((8, NP), jnp.float32),
    )(x, Wg8, bg8)

    Wc8 = jnp.zeros((8, HDIM), jnp.float32).at[:GDIM].set(Wc)
    out = pl.pallas_call(
        _tc_final_body,
        grid=(GRID,),
        in_specs=[pl.BlockSpec((8, BN), lambda j: (0, j)),
                  pl.BlockSpec((1, BN), lambda j: (0, j)),
                  pl.BlockSpec((1, BN), lambda j: (0, j)),
                  pl.BlockSpec((1, BN), lambda j: (0, j)),
                  pl.BlockSpec((1, BN), lambda j: (0, j)),
                  pl.BlockSpec((8, HDIM), lambda j: (0, 0)),
                  pl.BlockSpec((1, HDIM), lambda j: (0, 0))],
        out_specs=pl.BlockSpec((1, HDIM), lambda j: (0, 0)),
        out_shape=jax.ShapeDtypeStruct((1, HDIM), jnp.float32),
        scratch_shapes=[pltpu.VMEM((8, 128), jnp.float32)],
    )(gt,
      outd_a.reshape(1, NP), outd_b.reshape(1, NP),
      t_a.reshape(1, NP), t_b.reshape(1, NP),
      Wc8, bc.reshape(1, HDIM))

    return out


# speedup vs baseline: 47.2780x; 2.4807x over previous
import functools

import jax
import jax.numpy as jnp
from jax import lax
from jax.experimental import pallas as pl
from jax.experimental.pallas import tpu as pltpu
from jax.experimental.pallas import tpu_sc as plsc

N = 100000
E = 1600000
D = 128
GDIM = 7
HDIM = 256

BN = 2048
GRID = 49
NP = GRID * BN
NSUB = 16
SLICE = NP // NSUB

CH = 128
ROWS = 13
OUTER = 30
NCHUNK = E // CH
BASE_CHUNKS = 390
EXTRA_TILES = NCHUNK - 32 * BASE_CHUNKS

@functools.lru_cache(maxsize=None)
def _get_mesh():
    return plsc.VectorSubcoreMesh(core_axis_name="c", subcore_axis_name="s")


def _zero_slice(zero_v, shared, s):
    pltpu.sync_copy(zero_v, shared.at[pl.ds(s * SLICE, SLICE)])


def _tile_chunk_start(w):
    return jnp.where(w < EXTRA_TILES,
                     w * (BASE_CHUNKS + 1),
                     EXTRA_TILES * (BASE_CHUNKS + 1) + (w - EXTRA_TILES) * BASE_CHUNKS)


def _sc_hist(edge_hbm, outd_a, outd_b, ind_a, ind_b,
             src_v, dst_v, ones_v, zero_v, acc_out, acc_in, lsem, ssem):
    c = lax.axis_index("c")
    s = lax.axis_index("s")
    w = c * NSUB + s

    def fill_ones(i, _):
        ones_v[pl.ds(i * 16, 16)] = jnp.ones((16,), jnp.float32)
        return _
    lax.fori_loop(0, CH // 16, fill_ones, None)

    def fill_zero(i, _):
        zero_v[pl.ds(i * 16, 16)] = jnp.zeros((16,), jnp.float32)
        return _
    lax.fori_loop(0, SLICE // 16, fill_zero, None)

    _zero_slice(zero_v, acc_out, s)
    _zero_slice(zero_v, acc_in, s)
    plsc.subcore_barrier()

    start = _tile_chunk_start(w)

    def load(o, b):
        row0 = start + o * ROWS
        pltpu.async_copy(edge_hbm.at[0, pl.ds(row0, ROWS)], src_v.at[b], lsem)
        pltpu.async_copy(edge_hbm.at[1, pl.ds(row0, ROWS)], dst_v.at[b], lsem)

    def drain_load(b):
        pltpu.make_async_copy(edge_hbm.at[0, pl.ds(0, ROWS)], src_v.at[b], lsem).wait()
        pltpu.make_async_copy(edge_hbm.at[1, pl.ds(0, ROWS)], dst_v.at[b], lsem).wait()

    load(0, 0)

    def outer(o, _):
        b = lax.rem(o, 2)
        drain_load(b)

        @pl.when(o + 1 < OUTER)
        def _next():
            load(o + 1, 1 - b)

        descs = []
        for j in range(ROWS):
            descs.append(pltpu.async_copy(
                ones_v, acc_out.at[src_v.at[b, j]], ssem, add=True))
            descs.append(pltpu.async_copy(
                ones_v, acc_in.at[dst_v.at[b, j]], ssem, add=True))
        for d in descs:
            d.wait()
        return _
    lax.fori_loop(0, OUTER, outer, None)

    @pl.when(w < EXTRA_TILES)
    def _extra():
        row = start + BASE_CHUNKS
        pltpu.sync_copy(edge_hbm.at[0, pl.ds(row, 1)], src_v.at[0, pl.ds(0, 1)])
        pltpu.sync_copy(edge_hbm.at[1, pl.ds(row, 1)], dst_v.at[0, pl.ds(0, 1)])
        pltpu.sync_copy(ones_v, acc_out.at[src_v.at[0, 0]], add=True)
        pltpu.sync_copy(ones_v, acc_in.at[dst_v.at[0, 0]], add=True)

    plsc.subcore_barrier()
    off = s * SLICE

    @pl.when(c == 0)
    def _w0():
        pltpu.sync_copy(acc_out.at[pl.ds(off, SLICE)], outd_a.at[pl.ds(off, SLICE)])
        pltpu.sync_copy(acc_in.at[pl.ds(off, SLICE)], ind_a.at[pl.ds(off, SLICE)])

    @pl.when(c == 1)
    def _w1():
        pltpu.sync_copy(acc_out.at[pl.ds(off, SLICE)], outd_b.at[pl.ds(off, SLICE)])
        pltpu.sync_copy(acc_in.at[pl.ds(off, SLICE)], ind_b.at[pl.ds(off, SLICE)])


def _sc_edge(edge_hbm, q_hbm, t_a, t_b,
             src_v, dst_v, val_v, zero_v, acc_t, q_spm, lsem, gsem, ssem):
    c = lax.axis_index("c")
    s = lax.axis_index("s")
    w = c * NSUB + s

    def fill_zero(i, _):
        zero_v[pl.ds(i * 16, 16)] = jnp.zeros((16,), jnp.float32)
        return _
    lax.fori_loop(0, SLICE // 16, fill_zero, None)

    _zero_slice(zero_v, acc_t, s)
    pltpu.sync_copy(q_hbm.at[pl.ds(s * SLICE, SLICE)],
                    q_spm.at[pl.ds(s * SLICE, SLICE)])
    plsc.subcore_barrier()

    start = _tile_chunk_start(w)

    def load(o, b):
        row0 = start + o * ROWS
        pltpu.async_copy(edge_hbm.at[0, pl.ds(row0, ROWS)], src_v.at[b], lsem)
        pltpu.async_copy(edge_hbm.at[1, pl.ds(row0, ROWS)], dst_v.at[b], lsem)

    def drain_load(b):
        pltpu.make_async_copy(edge_hbm.at[0, pl.ds(0, ROWS)], src_v.at[b], lsem).wait()
        pltpu.make_async_copy(edge_hbm.at[1, pl.ds(0, ROWS)], dst_v.at[b], lsem).wait()

    load(0, 0)

    def outer(o, _):
        b = lax.rem(o, 2)
        drain_load(b)

        @pl.when(o + 1 < OUTER)
        def _next():
            load(o + 1, 1 - b)

        gd = []
        for j in range(ROWS):
            gd.append(pltpu.async_copy(
                q_hbm.at[dst_v.at[b, j]], val_v.at[j], gsem))
        for d in gd:
            d.wait()
        sd = []
        for j in range(ROWS):
            sd.append(pltpu.async_copy(
                val_v.at[j], acc_t.at[src_v.at[b, j]], ssem, add=True))
        for d in sd:
            d.wait()
        return _
    lax.fori_loop(0, OUTER, outer, None)

    @pl.when(w < EXTRA_TILES)
    def _extra():
        row = start + BASE_CHUNKS
        pltpu.sync_copy(edge_hbm.at[0, pl.ds(row, 1)], src_v.at[0, pl.ds(0, 1)])
        pltpu.sync_copy(edge_hbm.at[1, pl.ds(row, 1)], dst_v.at[0, pl.ds(0, 1)])
        pltpu.sync_copy(q_spm.at[dst_v.at[0, 0]], val_v.at[0])
        pltpu.sync_copy(val_v.at[0], acc_t.at[src_v.at[0, 0]], add=True)

    plsc.subcore_barrier()
    off = s * SLICE

    @pl.when(c == 0)
    def _w0():
        pltpu.sync_copy(acc_t.at[pl.ds(off, SLICE)], t_a.at[pl.ds(off, SLICE)])

    @pl.when(c == 1)
    def _w1():
        pltpu.sync_copy(acc_t.at[pl.ds(off, SLICE)], t_b.at[pl.ds(off, SLICE)])


def _tc_gate_body(x_ref, wg_ref, bg_ref, gt_ref):
    z = lax.dot_general(wg_ref[...], x_ref[...],
                        (((1,), (1,)), ((), ())),
                        preferred_element_type=jnp.float32)
    z = z + bg_ref[:, 0:1]
    gt_ref[...] = 1.0 / (1.0 + jnp.exp(-z))


def _tc_q_body(ia_ref, ib_ref, q_ref):
    d = jnp.maximum(ia_ref[...] + ib_ref[...], 1.0)
    q_ref[...] = lax.rsqrt(d)


def _tc_final_body(gt_ref, oa_ref, ob_ref, ta_ref, tb_ref, wc_ref, bc_ref,
                   out_ref, acc_ref):
    j = pl.program_id(0)

    @pl.when(j == 0)
    def _init():
        acc_ref[...] = jnp.zeros((8, 128), jnp.float32)

    outd = jnp.maximum(oa_ref[...] + ob_ref[...], 1.0)
    cvec = lax.rsqrt(outd) * (ta_ref[...] + tb_ref[...])
    node = j * BN + lax.broadcasted_iota(jnp.int32, (1, BN), 1)
    prod = jnp.where(node < N, gt_ref[...] * cvec, 0.0)
    acc = acc_ref[...]
    for k in range(BN // 128):
        acc = acc + prod[:, k * 128:(k + 1) * 128]
    acc_ref[...] = acc

    @pl.when(j == GRID - 1)
    def _fin():
        s8 = jnp.sum(acc_ref[...], axis=1, keepdims=True)
        out = lax.dot_general(s8, wc_ref[...],
                              (((0,), (0,)), ((), ())),
                              preferred_element_type=jnp.float32)
        out_ref[...] = out + float(N) * bc_ref[...]


@functools.lru_cache(maxsize=None)
def _get_sc_kernels():
    mesh = _get_mesh()
    params = pltpu.CompilerParams(use_tc_tiling_on_sc=False)
    hist = functools.partial(
        pl.kernel,
        mesh=mesh,
        compiler_params=params,
        out_type=[jax.ShapeDtypeStruct((NP,), jnp.float32) for _ in range(4)],
        scratch_types=[
            pltpu.VMEM((2, ROWS, CH), jnp.int32),
            pltpu.VMEM((2, ROWS, CH), jnp.int32),
            pltpu.VMEM((CH,), jnp.float32),
            pltpu.VMEM((SLICE,), jnp.float32),
            pltpu.VMEM_SHARED((NP,), jnp.float32),
            pltpu.VMEM_SHARED((NP,), jnp.float32),
            pltpu.SemaphoreType.DMA,
            pltpu.SemaphoreType.DMA,
        ],
    )(_sc_hist)
    edge = functools.partial(
        pl.kernel,
        mesh=mesh,
        compiler_params=params,
        out_type=[jax.ShapeDtypeStruct((NP,), jnp.float32) for _ in range(2)],
        scratch_types=[
            pltpu.VMEM((2, ROWS, CH), jnp.int32),
            pltpu.VMEM((2, ROWS, CH), jnp.int32),
            pltpu.VMEM((ROWS, CH), jnp.float32),
            pltpu.VMEM((SLICE,), jnp.float32),
            pltpu.VMEM_SHARED((NP,), jnp.float32),
            pltpu.VMEM_SHARED((NP,), jnp.float32),
            pltpu.SemaphoreType.DMA,
            pltpu.SemaphoreType.DMA,
            pltpu.SemaphoreType.DMA,
        ],
    )(_sc_edge)
    return hist, edge


def kernel(x, edge_index, Wg, bg, Wc, bc):
    edge3 = edge_index.reshape(2, NCHUNK, CH)
    sc_hist, sc_edge = _get_sc_kernels()

    outd_a, outd_b, ind_a, ind_b = sc_hist(edge3)

    q = pl.pallas_call(
        _tc_q_body,
        grid=(GRID,),
        in_specs=[pl.BlockSpec((1, BN), lambda j: (0, j)),
                  pl.BlockSpec((1, BN), lambda j: (0, j))],
        out_specs=pl.BlockSpec((1, BN), lambda j: (0, j)),
        out_shape=jax.ShapeDtypeStruct((1, NP), jnp.float32),
    )(ind_a.reshape(1, NP), ind_b.reshape(1, NP))

    t_a, t_b = sc_edge(edge3, q.reshape(NP))

    Wg8 = jnp.zeros((8, D), jnp.float32).at[:GDIM].set(Wg)
    bg8 = jnp.broadcast_to(
        jnp.zeros((8,), jnp.float32).at[:GDIM].set(bg)[:, None], (8, 128))
    gt = pl.pallas_call(
        _tc_gate_body,
        grid=(GRID,),
        in_specs=[pl.BlockSpec((BN, D), lambda j: (j, 0)),
                  pl.BlockSpec((8, D), lambda j: (0, 0)),
                  pl.BlockSpec((8, 128), lambda j: (0, 0))],
        out_specs=pl.BlockSpec((8, BN), lambda j: (0, j)),
        out_shape=jax.ShapeDtypeStruct((8, NP), jnp.float32),
    )(x, Wg8, bg8)

    Wc8 = jnp.zeros((8, HDIM), jnp.float32).at[:GDIM].set(Wc)
    out = pl.pallas_call(
        _tc_final_body,
        grid=(GRID,),
        in_specs=[pl.BlockSpec((8, BN), lambda j: (0, j)),
                  pl.BlockSpec((1, BN), lambda j: (0, j)),
                  pl.BlockSpec((1, BN), lambda j: (0, j)),
                  pl.BlockSpec((1, BN), lambda j: (0, j)),
                  pl.BlockSpec((1, BN), lambda j: (0, j)),
                  pl.BlockSpec((8, HDIM), lambda j: (0, 0)),
                  pl.BlockSpec((1, HDIM), lambda j: (0, 0))],
        out_specs=pl.BlockSpec((1, HDIM), lambda j: (0, 0)),
        out_shape=jax.ShapeDtypeStruct((1, HDIM), jnp.float32),
        scratch_shapes=[pltpu.VMEM((8, 128), jnp.float32)],
    )(gt,
      outd_a.reshape(1, NP), outd_b.reshape(1, NP),
      t_a.reshape(1, NP), t_b.reshape(1, NP),
      Wc8, bc.reshape(1, HDIM))

    return out


# speedup vs baseline: 59.2043x; 1.2523x over previous
import functools

import jax
import jax.numpy as jnp
from jax import lax
from jax.experimental import pallas as pl
from jax.experimental.pallas import tpu as pltpu
from jax.experimental.pallas import tpu_sc as plsc

N = 100000
E = 1600000
D = 128
GDIM = 7
HDIM = 256

BN = 2048
GRID = 49
NP = GRID * BN
NSUB = 16
SLICE = NP // NSUB

CH = 128
ROWS = 13
OUTER = 30
NCHUNK = E // CH
BASE_CHUNKS = 390
EXTRA_TILES = NCHUNK - 32 * BASE_CHUNKS

@functools.lru_cache(maxsize=None)
def _get_mesh():
    return plsc.VectorSubcoreMesh(core_axis_name="c", subcore_axis_name="s")


def _zero_slice(zero_v, shared, s):
    pltpu.sync_copy(zero_v, shared.at[pl.ds(s * SLICE, SLICE)])


def _tile_chunk_start(w):
    return jnp.where(w < EXTRA_TILES,
                     w * (BASE_CHUNKS + 1),
                     EXTRA_TILES * (BASE_CHUNKS + 1) + (w - EXTRA_TILES) * BASE_CHUNKS)


def _sc_hist(edge_hbm, outd_a, outd_b, ind_a, ind_b,
             src_v, dst_v, ones_v, zero_v, acc_out, acc_in, lsem, ssem):
    c = lax.axis_index("c")
    s = lax.axis_index("s")
    w = c * NSUB + s

    def fill_ones(i, _):
        ones_v[pl.ds(i * 16, 16)] = jnp.ones((16,), jnp.float32)
        return _
    lax.fori_loop(0, CH // 16, fill_ones, None)

    def fill_zero(i, _):
        zero_v[pl.ds(i * 16, 16)] = jnp.zeros((16,), jnp.float32)
        return _
    lax.fori_loop(0, SLICE // 16, fill_zero, None)

    _zero_slice(zero_v, acc_out, s)
    _zero_slice(zero_v, acc_in, s)
    plsc.subcore_barrier()

    start = _tile_chunk_start(w)

    def load(o, b):
        row0 = start + o * ROWS
        pltpu.async_copy(edge_hbm.at[0, pl.ds(row0, ROWS)], src_v.at[b], lsem)
        pltpu.async_copy(edge_hbm.at[1, pl.ds(row0, ROWS)], dst_v.at[b], lsem)

    def drain_load(b):
        pltpu.make_async_copy(edge_hbm.at[0, pl.ds(0, ROWS)], src_v.at[b], lsem).wait()
        pltpu.make_async_copy(edge_hbm.at[1, pl.ds(0, ROWS)], dst_v.at[b], lsem).wait()

    load(0, 0)

    def outer(o, _):
        b = lax.rem(o, 2)
        drain_load(b)

        @pl.when(o + 1 < OUTER)
        def _next():
            load(o + 1, 1 - b)

        descs = []
        for j in range(ROWS):
            descs.append(pltpu.async_copy(
                ones_v, acc_out.at[src_v.at[b, j]], ssem, add=True))
            descs.append(pltpu.async_copy(
                ones_v, acc_in.at[dst_v.at[b, j]], ssem, add=True))
        for d in descs:
            d.wait()
        return _
    lax.fori_loop(0, OUTER, outer, None)

    @pl.when(w < EXTRA_TILES)
    def _extra():
        row = start + BASE_CHUNKS
        pltpu.sync_copy(edge_hbm.at[0, pl.ds(row, 1)], src_v.at[0, pl.ds(0, 1)])
        pltpu.sync_copy(edge_hbm.at[1, pl.ds(row, 1)], dst_v.at[0, pl.ds(0, 1)])
        pltpu.sync_copy(ones_v, acc_out.at[src_v.at[0, 0]], add=True)
        pltpu.sync_copy(ones_v, acc_in.at[dst_v.at[0, 0]], add=True)

    plsc.subcore_barrier()
    off = s * SLICE

    @pl.when(c == 0)
    def _w0():
        pltpu.sync_copy(acc_out.at[pl.ds(off, SLICE)], outd_a.at[pl.ds(off, SLICE)])
        pltpu.sync_copy(acc_in.at[pl.ds(off, SLICE)], ind_a.at[pl.ds(off, SLICE)])

    @pl.when(c == 1)
    def _w1():
        pltpu.sync_copy(acc_out.at[pl.ds(off, SLICE)], outd_b.at[pl.ds(off, SLICE)])
        pltpu.sync_copy(acc_in.at[pl.ds(off, SLICE)], ind_b.at[pl.ds(off, SLICE)])


def _rsqrt16(x):
    i = lax.bitcast_convert_type(x, jnp.int32)
    i = jnp.int32(0x5F3759DF) - lax.shift_right_logical(i, 1)
    y = lax.bitcast_convert_type(i, jnp.float32)
    for _ in range(3):
        y = y * (1.5 - 0.5 * x * y * y)
    return y


QCH = SLICE // 4


def _sc_edge(edge_hbm, ia_hbm, ib_hbm, t_a, t_b, q_a, q_b,
             src_v, dst_v, val_v, iva_v, ivb_v, qsl_v, q_vmem,
             acc_t, lsem, ssem):
    c = lax.axis_index("c")
    s = lax.axis_index("s")
    w = c * NSUB + s

    def fill_zero(i, _):
        qsl_v[pl.ds(i * 16, 16)] = jnp.zeros((16,), jnp.float32)
        return _
    lax.fori_loop(0, QCH // 16, fill_zero, None)
    for qc in range(4):
        pltpu.sync_copy(qsl_v, acc_t.at[pl.ds(s * SLICE + qc * QCH, QCH)])

    for qc in range(4):
        off = s * SLICE + qc * QCH
        pltpu.sync_copy(ia_hbm.at[pl.ds(off, QCH)], iva_v)
        pltpu.sync_copy(ib_hbm.at[pl.ds(off, QCH)], ivb_v)

        def qbody(i, _):
            d = jnp.maximum(iva_v[pl.ds(i * 16, 16)] + ivb_v[pl.ds(i * 16, 16)],
                            1.0)
            qsl_v[pl.ds(i * 16, 16)] = _rsqrt16(d)
            return _
        lax.fori_loop(0, QCH // 16, qbody, None)

        @pl.when(c == 0)
        def _q0():
            pltpu.sync_copy(qsl_v, q_a.at[pl.ds(off, QCH)])

        @pl.when(c == 1)
        def _q1():
            pltpu.sync_copy(qsl_v, q_b.at[pl.ds(off, QCH)])

    plsc.subcore_barrier()

    @pl.when(c == 0)
    def _l0():
        pltpu.sync_copy(q_a, q_vmem)

    @pl.when(c == 1)
    def _l1():
        pltpu.sync_copy(q_b, q_vmem)

    start = _tile_chunk_start(w)

    def load(o, b):
        row0 = start + o * ROWS
        pltpu.async_copy(edge_hbm.at[0, pl.ds(row0, ROWS)], src_v.at[b], lsem)
        pltpu.async_copy(edge_hbm.at[1, pl.ds(row0, ROWS)], dst_v.at[b], lsem)

    def drain_load(b):
        pltpu.make_async_copy(edge_hbm.at[0, pl.ds(0, ROWS)], src_v.at[b], lsem).wait()
        pltpu.make_async_copy(edge_hbm.at[1, pl.ds(0, ROWS)], dst_v.at[b], lsem).wait()

    load(0, 0)

    def outer(o, _):
        b = lax.rem(o, 2)
        drain_load(b)

        @pl.when(o + 1 < OUTER)
        def _next():
            load(o + 1, 1 - b)

        for j in range(ROWS):
            for k in range(CH // 16):
                idx16 = dst_v[b, j, pl.ds(k * 16, 16)]
                val_v[j, pl.ds(k * 16, 16)] = plsc.load_gather(q_vmem, [idx16])
        sd = []
        for j in range(ROWS):
            sd.append(pltpu.async_copy(
                val_v.at[j], acc_t.at[src_v.at[b, j]], ssem, add=True))
        for d in sd:
            d.wait()
        return _
    lax.fori_loop(0, OUTER, outer, None)

    @pl.when(w < EXTRA_TILES)
    def _extra():
        row = start + BASE_CHUNKS
        pltpu.sync_copy(edge_hbm.at[0, pl.ds(row, 1)], src_v.at[0, pl.ds(0, 1)])
        pltpu.sync_copy(edge_hbm.at[1, pl.ds(row, 1)], dst_v.at[0, pl.ds(0, 1)])
        for k in range(CH // 16):
            idx16 = dst_v[0, 0, pl.ds(k * 16, 16)]
            val_v[0, pl.ds(k * 16, 16)] = plsc.load_gather(q_vmem, [idx16])
        pltpu.sync_copy(val_v.at[0], acc_t.at[src_v.at[0, 0]], add=True)

    plsc.subcore_barrier()
    off = s * SLICE

    @pl.when(c == 0)
    def _w0():
        pltpu.sync_copy(acc_t.at[pl.ds(off, SLICE)], t_a.at[pl.ds(off, SLICE)])

    @pl.when(c == 1)
    def _w1():
        pltpu.sync_copy(acc_t.at[pl.ds(off, SLICE)], t_b.at[pl.ds(off, SLICE)])


def _tc_final_body(x_ref, wg_ref, bg_ref, oa_ref, ob_ref, ta_ref, tb_ref,
                   wc_ref, bc_ref, out_ref, acc_ref):
    j = pl.program_id(0)

    @pl.when(j == 0)
    def _init():
        acc_ref[...] = jnp.zeros((8, 128), jnp.float32)

    z = lax.dot_general(wg_ref[...], x_ref[...],
                        (((1,), (1,)), ((), ())),
                        preferred_element_type=jnp.float32)
    z = z + bg_ref[:, 0:1]
    gt = 1.0 / (1.0 + jnp.exp(-z))

    outd = jnp.maximum(oa_ref[...] + ob_ref[...], 1.0)
    cvec = lax.rsqrt(outd) * (ta_ref[...] + tb_ref[...])
    node = j * BN + lax.broadcasted_iota(jnp.int32, (1, BN), 1)
    prod = jnp.where(node < N, gt * cvec, 0.0)
    acc = acc_ref[...]
    for k in range(BN // 128):
        acc = acc + prod[:, k * 128:(k + 1) * 128]
    acc_ref[...] = acc

    @pl.when(j == GRID - 1)
    def _fin():
        s8 = jnp.sum(acc_ref[...], axis=1, keepdims=True)
        out = lax.dot_general(s8, wc_ref[...],
                              (((0,), (0,)), ((), ())),
                              preferred_element_type=jnp.float32)
        out_ref[...] = out + float(N) * bc_ref[...]


@functools.lru_cache(maxsize=None)
def _get_sc_kernels():
    mesh = _get_mesh()
    params = pltpu.CompilerParams(use_tc_tiling_on_sc=False,
                                  needs_layout_passes=False)
    hist = functools.partial(
        pl.kernel,
        mesh=mesh,
        compiler_params=params,
        out_type=[jax.ShapeDtypeStruct((NP,), jnp.float32) for _ in range(4)],
        scratch_types=[
            pltpu.VMEM((2, ROWS, CH), jnp.int32),
            pltpu.VMEM((2, ROWS, CH), jnp.int32),
            pltpu.VMEM((CH,), jnp.float32),
            pltpu.VMEM((SLICE,), jnp.float32),
            pltpu.VMEM_SHARED((NP,), jnp.float32),
            pltpu.VMEM_SHARED((NP,), jnp.float32),
            pltpu.SemaphoreType.DMA,
            pltpu.SemaphoreType.DMA,
        ],
    )(_sc_hist)
    edge = functools.partial(
        pl.kernel,
        mesh=mesh,
        compiler_params=params,
        out_type=[jax.ShapeDtypeStruct((NP,), jnp.float32) for _ in range(4)],
        scratch_types=[
            pltpu.VMEM((2, ROWS, CH), jnp.int32),
            pltpu.VMEM((2, ROWS, CH), jnp.int32),
            pltpu.VMEM((ROWS, CH), jnp.float32),
            pltpu.VMEM((QCH,), jnp.float32),
            pltpu.VMEM((QCH,), jnp.float32),
            pltpu.VMEM((QCH,), jnp.float32),
            pltpu.VMEM((NP,), jnp.float32),
            pltpu.VMEM_SHARED((NP,), jnp.float32),
            pltpu.SemaphoreType.DMA,
            pltpu.SemaphoreType.DMA,
        ],
    )(_sc_edge)
    return hist, edge


def kernel(x, edge_index, Wg, bg, Wc, bc):
    edge3 = edge_index.reshape(2, NCHUNK, CH)
    sc_hist, sc_edge = _get_sc_kernels()

    outd_a, outd_b, ind_a, ind_b = sc_hist(edge3)
    t_a, t_b, _, _ = sc_edge(edge3, ind_a, ind_b)

    Wg8 = jnp.zeros((8, D), jnp.float32).at[:GDIM].set(Wg)
    bg8 = jnp.broadcast_to(
        jnp.zeros((8,), jnp.float32).at[:GDIM].set(bg)[:, None], (8, 128))
    Wc8 = jnp.zeros((8, HDIM), jnp.float32).at[:GDIM].set(Wc)
    out = pl.pallas_call(
        _tc_final_body,
        grid=(GRID,),
        in_specs=[pl.BlockSpec((BN, D), lambda j: (j, 0)),
                  pl.BlockSpec((8, D), lambda j: (0, 0)),
                  pl.BlockSpec((8, 128), lambda j: (0, 0)),
                  pl.BlockSpec((1, BN), lambda j: (0, j)),
                  pl.BlockSpec((1, BN), lambda j: (0, j)),
                  pl.BlockSpec((1, BN), lambda j: (0, j)),
                  pl.BlockSpec((1, BN), lambda j: (0, j)),
                  pl.BlockSpec((8, HDIM), lambda j: (0, 0)),
                  pl.BlockSpec((1, HDIM), lambda j: (0, 0))],
        out_specs=pl.BlockSpec((1, HDIM), lambda j: (0, 0)),
        out_shape=jax.ShapeDtypeStruct((1, HDIM), jnp.float32),
        scratch_shapes=[pltpu.VMEM((8, 128), jnp.float32)],
    )(x, Wg8, bg8,
      outd_a.reshape(1, NP), outd_b.reshape(1, NP),
      t_a.reshape(1, NP), t_b.reshape(1, NP),
      Wc8, bc.reshape(1, HDIM))

    return out


# speedup vs baseline: 59.2628x; 1.0010x over previous
import functools

import jax
import jax.numpy as jnp
from jax import lax
from jax.experimental import pallas as pl
from jax.experimental.pallas import tpu as pltpu
from jax.experimental.pallas import tpu_sc as plsc

N = 100000
E = 1600000
D = 128
GDIM = 7
HDIM = 256

BN = 2048
GRID = 49
NP = GRID * BN
NSUB = 16
SLICE = NP // NSUB

CH = 128
ROWS = 13
BIG = ROWS * CH
OUTER = 30
NCHUNK = E // CH
BASE_CHUNKS = 390
EXTRA_TILES = NCHUNK - 32 * BASE_CHUNKS

@functools.lru_cache(maxsize=None)
def _get_mesh():
    return plsc.VectorSubcoreMesh(core_axis_name="c", subcore_axis_name="s")


def _zero_slice(zero_v, shared, s):
    pltpu.sync_copy(zero_v, shared.at[pl.ds(s * SLICE, SLICE)])


def _tile_chunk_start(w):
    return jnp.where(w < EXTRA_TILES,
                     w * (BASE_CHUNKS + 1),
                     EXTRA_TILES * (BASE_CHUNKS + 1) + (w - EXTRA_TILES) * BASE_CHUNKS)


def _sc_hist(edge_hbm, outd_a, outd_b, ind_a, ind_b,
             src_v, dst_v, ones_v, zero_v, acc_out, acc_in, lsem, ssem):
    c = lax.axis_index("c")
    s = lax.axis_index("s")
    w = c * NSUB + s

    def fill_ones(i, _):
        ones_v[0, pl.ds(i * 16, 16)] = jnp.ones((16,), jnp.float32)
        return _
    lax.fori_loop(0, BIG // 16, fill_ones, None)

    def fill_zero(i, _):
        zero_v[pl.ds(i * 16, 16)] = jnp.zeros((16,), jnp.float32)
        return _
    lax.fori_loop(0, SLICE // 16, fill_zero, None)

    _zero_slice(zero_v, acc_out, s)
    _zero_slice(zero_v, acc_in, s)
    plsc.subcore_barrier()

    estart = _tile_chunk_start(w) * CH

    def load(o, b):
        e0 = estart + o * BIG
        pltpu.async_copy(edge_hbm.at[0, pl.ds(e0, BIG)], src_v.at[b, 0], lsem)
        pltpu.async_copy(edge_hbm.at[1, pl.ds(e0, BIG)], dst_v.at[b, 0], lsem)

    def drain_load(b):
        pltpu.make_async_copy(edge_hbm.at[0, pl.ds(0, BIG)], src_v.at[b, 0], lsem).wait()
        pltpu.make_async_copy(edge_hbm.at[1, pl.ds(0, BIG)], dst_v.at[b, 0], lsem).wait()

    load(0, 0)

    def outer(o, _):
        b = lax.rem(o, 2)
        drain_load(b)

        @pl.when(o + 1 < OUTER)
        def _next():
            load(o + 1, 1 - b)

        d1 = pltpu.async_copy(ones_v.at[0], acc_out.at[src_v.at[b, 0]], ssem,
                              add=True)
        d2 = pltpu.async_copy(ones_v.at[0], acc_in.at[dst_v.at[b, 0]], ssem,
                              add=True)
        d1.wait()
        d2.wait()
        return _
    lax.fori_loop(0, OUTER, outer, None)

    @pl.when(w < EXTRA_TILES)
    def _extra():
        e0 = estart + BASE_CHUNKS * CH
        pltpu.sync_copy(edge_hbm.at[0, pl.ds(e0, CH)], src_v.at[0, 0, pl.ds(0, CH)])
        pltpu.sync_copy(edge_hbm.at[1, pl.ds(e0, CH)], dst_v.at[0, 0, pl.ds(0, CH)])
        pltpu.sync_copy(ones_v.at[0, pl.ds(0, CH)],
                        acc_out.at[src_v.at[0, 0, pl.ds(0, CH)]], add=True)
        pltpu.sync_copy(ones_v.at[0, pl.ds(0, CH)],
                        acc_in.at[dst_v.at[0, 0, pl.ds(0, CH)]], add=True)

    plsc.subcore_barrier()
    off = s * SLICE

    @pl.when(c == 0)
    def _w0():
        pltpu.sync_copy(acc_out.at[pl.ds(off, SLICE)], outd_a.at[pl.ds(off, SLICE)])
        pltpu.sync_copy(acc_in.at[pl.ds(off, SLICE)], ind_a.at[pl.ds(off, SLICE)])

    @pl.when(c == 1)
    def _w1():
        pltpu.sync_copy(acc_out.at[pl.ds(off, SLICE)], outd_b.at[pl.ds(off, SLICE)])
        pltpu.sync_copy(acc_in.at[pl.ds(off, SLICE)], ind_b.at[pl.ds(off, SLICE)])


def _rsqrt16(x):
    i = lax.bitcast_convert_type(x, jnp.int32)
    i = jnp.int32(0x5F3759DF) - lax.shift_right_logical(i, 1)
    y = lax.bitcast_convert_type(i, jnp.float32)
    for _ in range(3):
        y = y * (1.5 - 0.5 * x * y * y)
    return y


QCH = SLICE // 4


def _sc_edge(edge_hbm, ia_hbm, ib_hbm, t_a, t_b, q_a, q_b,
             src_v, dst_v, val_v, iva_v, ivb_v, qsl_v, q_vmem,
             acc_t, lsem, ssem):
    c = lax.axis_index("c")
    s = lax.axis_index("s")
    w = c * NSUB + s

    def fill_zero(i, _):
        qsl_v[pl.ds(i * 16, 16)] = jnp.zeros((16,), jnp.float32)
        return _
    lax.fori_loop(0, QCH // 16, fill_zero, None)
    for qc in range(4):
        pltpu.sync_copy(qsl_v, acc_t.at[pl.ds(s * SLICE + qc * QCH, QCH)])

    for qc in range(4):
        off = s * SLICE + qc * QCH
        pltpu.sync_copy(ia_hbm.at[pl.ds(off, QCH)], iva_v)
        pltpu.sync_copy(ib_hbm.at[pl.ds(off, QCH)], ivb_v)

        def qbody(i, _):
            d = jnp.maximum(iva_v[pl.ds(i * 16, 16)] + ivb_v[pl.ds(i * 16, 16)],
                            1.0)
            qsl_v[pl.ds(i * 16, 16)] = _rsqrt16(d)
            return _
        lax.fori_loop(0, QCH // 16, qbody, None)

        @pl.when(c == 0)
        def _q0():
            pltpu.sync_copy(qsl_v, q_a.at[pl.ds(off, QCH)])

        @pl.when(c == 1)
        def _q1():
            pltpu.sync_copy(qsl_v, q_b.at[pl.ds(off, QCH)])

    plsc.subcore_barrier()

    @pl.when(c == 0)
    def _l0():
        pltpu.sync_copy(q_a, q_vmem)

    @pl.when(c == 1)
    def _l1():
        pltpu.sync_copy(q_b, q_vmem)

    estart = _tile_chunk_start(w) * CH

    def load(o, b):
        e0 = estart + o * BIG
        pltpu.async_copy(edge_hbm.at[0, pl.ds(e0, BIG)], src_v.at[b, 0], lsem)
        pltpu.async_copy(edge_hbm.at[1, pl.ds(e0, BIG)], dst_v.at[b, 0], lsem)

    def drain_load(b):
        pltpu.make_async_copy(edge_hbm.at[0, pl.ds(0, BIG)], src_v.at[b, 0], lsem).wait()
        pltpu.make_async_copy(edge_hbm.at[1, pl.ds(0, BIG)], dst_v.at[b, 0], lsem).wait()

    load(0, 0)

    def outer(o, _):
        b = lax.rem(o, 2)
        drain_load(b)

        @pl.when(o + 1 < OUTER)
        def _next():
            load(o + 1, 1 - b)

        for k in range(BIG // 16):
            idx16 = dst_v[b, 0, pl.ds(k * 16, 16)]
            val_v[0, pl.ds(k * 16, 16)] = plsc.load_gather(q_vmem, [idx16])
        pltpu.async_copy(val_v.at[0], acc_t.at[src_v.at[b, 0]], ssem,
                         add=True).wait()
        return _
    lax.fori_loop(0, OUTER, outer, None)

    @pl.when(w < EXTRA_TILES)
    def _extra():
        e0 = estart + BASE_CHUNKS * CH
        pltpu.sync_copy(edge_hbm.at[0, pl.ds(e0, CH)], src_v.at[0, 0, pl.ds(0, CH)])
        pltpu.sync_copy(edge_hbm.at[1, pl.ds(e0, CH)], dst_v.at[0, 0, pl.ds(0, CH)])
        for k in range(CH // 16):
            idx16 = dst_v[0, 0, pl.ds(k * 16, 16)]
            val_v[0, pl.ds(k * 16, 16)] = plsc.load_gather(q_vmem, [idx16])
        pltpu.sync_copy(val_v.at[0, pl.ds(0, CH)],
                        acc_t.at[src_v.at[0, 0, pl.ds(0, CH)]], add=True)

    plsc.subcore_barrier()
    off = s * SLICE

    @pl.when(c == 0)
    def _w0():
        pltpu.sync_copy(acc_t.at[pl.ds(off, SLICE)], t_a.at[pl.ds(off, SLICE)])

    @pl.when(c == 1)
    def _w1():
        pltpu.sync_copy(acc_t.at[pl.ds(off, SLICE)], t_b.at[pl.ds(off, SLICE)])


def _tc_final_body(x_ref, wg_ref, bg_ref, oa_ref, ob_ref, ta_ref, tb_ref,
                   wc_ref, bc_ref, out_ref, acc_ref):
    j = pl.program_id(0)

    @pl.when(j == 0)
    def _init():
        acc_ref[...] = jnp.zeros((8, 128), jnp.float32)

    z = lax.dot_general(wg_ref[...], x_ref[...],
                        (((1,), (1,)), ((), ())),
                        preferred_element_type=jnp.float32)
    z = z + bg_ref[:, 0:1]
    gt = 1.0 / (1.0 + jnp.exp(-z))

    outd = jnp.maximum(oa_ref[...] + ob_ref[...], 1.0)
    cvec = lax.rsqrt(outd) * (ta_ref[...] + tb_ref[...])
    node = j * BN + lax.broadcasted_iota(jnp.int32, (1, BN), 1)
    prod = jnp.where(node < N, gt * cvec, 0.0)
    acc = acc_ref[...]
    for k in range(BN // 128):
        acc = acc + prod[:, k * 128:(k + 1) * 128]
    acc_ref[...] = acc

    @pl.when(j == GRID - 1)
    def _fin():
        s8 = jnp.sum(acc_ref[...], axis=1, keepdims=True)
        out = lax.dot_general(s8, wc_ref[...],
                              (((0,), (0,)), ((), ())),
                              preferred_element_type=jnp.float32)
        out_ref[...] = out + float(N) * bc_ref[...]


@functools.lru_cache(maxsize=None)
def _get_sc_kernels():
    mesh = _get_mesh()
    params = pltpu.CompilerParams(use_tc_tiling_on_sc=False,
                                  needs_layout_passes=False)
    hist = functools.partial(
        pl.kernel,
        mesh=mesh,
        compiler_params=params,
        out_type=[jax.ShapeDtypeStruct((NP,), jnp.float32) for _ in range(4)],
        scratch_types=[
            pltpu.VMEM((2, 1, BIG), jnp.int32),
            pltpu.VMEM((2, 1, BIG), jnp.int32),
            pltpu.VMEM((1, BIG), jnp.float32),
            pltpu.VMEM((SLICE,), jnp.float32),
            pltpu.VMEM_SHARED((NP,), jnp.float32),
            pltpu.VMEM_SHARED((NP,), jnp.float32),
            pltpu.SemaphoreType.DMA,
            pltpu.SemaphoreType.DMA,
        ],
    )(_sc_hist)
    edge = functools.partial(
        pl.kernel,
        mesh=mesh,
        compiler_params=params,
        out_type=[jax.ShapeDtypeStruct((NP,), jnp.float32) for _ in range(4)],
        scratch_types=[
            pltpu.VMEM((2, 1, BIG), jnp.int32),
            pltpu.VMEM((2, 1, BIG), jnp.int32),
            pltpu.VMEM((1, BIG), jnp.float32),
            pltpu.VMEM((QCH,), jnp.float32),
            pltpu.VMEM((QCH,), jnp.float32),
            pltpu.VMEM((QCH,), jnp.float32),
            pltpu.VMEM((NP,), jnp.float32),
            pltpu.VMEM_SHARED((NP,), jnp.float32),
            pltpu.SemaphoreType.DMA,
            pltpu.SemaphoreType.DMA,
        ],
    )(_sc_edge)
    return hist, edge


def kernel(x, edge_index, Wg, bg, Wc, bc):
    sc_hist, sc_edge = _get_sc_kernels()

    outd_a, outd_b, ind_a, ind_b = sc_hist(edge_index)
    t_a, t_b, _, _ = sc_edge(edge_index, ind_a, ind_b)

    Wg8 = jnp.zeros((8, D), jnp.float32).at[:GDIM].set(Wg)
    bg8 = jnp.broadcast_to(
        jnp.zeros((8,), jnp.float32).at[:GDIM].set(bg)[:, None], (8, 128))
    Wc8 = jnp.zeros((8, HDIM), jnp.float32).at[:GDIM].set(Wc)
    out = pl.pallas_call(
        _tc_final_body,
        grid=(GRID,),
        in_specs=[pl.BlockSpec((BN, D), lambda j: (j, 0)),
                  pl.BlockSpec((8, D), lambda j: (0, 0)),
                  pl.BlockSpec((8, 128), lambda j: (0, 0)),
                  pl.BlockSpec((1, BN), lambda j: (0, j)),
                  pl.BlockSpec((1, BN), lambda j: (0, j)),
                  pl.BlockSpec((1, BN), lambda j: (0, j)),
                  pl.BlockSpec((1, BN), lambda j: (0, j)),
                  pl.BlockSpec((8, HDIM), lambda j: (0, 0)),
                  pl.BlockSpec((1, HDIM), lambda j: (0, 0))],
        out_specs=pl.BlockSpec((1, HDIM), lambda j: (0, 0)),
        out_shape=jax.ShapeDtypeStruct((1, HDIM), jnp.float32),
        scratch_shapes=[pltpu.VMEM((8, 128), jnp.float32)],
    )(x, Wg8, bg8,
      outd_a.reshape(1, NP), outd_b.reshape(1, NP),
      t_a.reshape(1, NP), t_b.reshape(1, NP),
      Wc8, bc.reshape(1, HDIM))

    return out


# speedup vs baseline: 63.9859x; 1.0797x over previous
import functools

import jax
import jax.numpy as jnp
from jax import lax
from jax.experimental import pallas as pl
from jax.experimental.pallas import tpu as pltpu
from jax.experimental.pallas import tpu_sc as plsc

N = 100000
E = 1600000
D = 128
GDIM = 7
HDIM = 256

BN = 2048
GRID = 49
NP = GRID * BN
NSUB = 16
SLICE = NP // NSUB

CH = 128
ROWS = 13
BIG = ROWS * CH
OUTER = 30
NCHUNK = E // CH
BASE_CHUNKS = 390
EXTRA_TILES = NCHUNK - 32 * BASE_CHUNKS

@functools.lru_cache(maxsize=None)
def _get_mesh():
    return plsc.VectorSubcoreMesh(core_axis_name="c", subcore_axis_name="s")


def _zero_slice(zero_v, shared, s):
    pltpu.sync_copy(zero_v, shared.at[pl.ds(s * SLICE, SLICE)])


def _tile_chunk_start(w):
    return jnp.where(w < EXTRA_TILES,
                     w * (BASE_CHUNKS + 1),
                     EXTRA_TILES * (BASE_CHUNKS + 1) + (w - EXTRA_TILES) * BASE_CHUNKS)


def _sc_hist(edge_hbm, outd_a, outd_b, ind_a, ind_b,
             src_v, dst_v, ones_v, zero_v, acc_out, acc_in, lsem, ssem):
    c = lax.axis_index("c")
    s = lax.axis_index("s")
    w = c * NSUB + s

    def fill_ones(i, _):
        ones_v[0, pl.ds(i * 16, 16)] = jnp.ones((16,), jnp.float32)
        return _
    lax.fori_loop(0, BIG // 16, fill_ones, None)

    def fill_zero(i, _):
        zero_v[pl.ds(i * 16, 16)] = jnp.zeros((16,), jnp.float32)
        return _
    lax.fori_loop(0, SLICE // 16, fill_zero, None)

    _zero_slice(zero_v, acc_out, s)
    _zero_slice(zero_v, acc_in, s)
    plsc.subcore_barrier()

    estart = _tile_chunk_start(w) * CH

    def load(o, b):
        e0 = estart + o * BIG
        pltpu.async_copy(edge_hbm.at[0, pl.ds(e0, BIG)], src_v.at[b, 0], lsem)
        pltpu.async_copy(edge_hbm.at[1, pl.ds(e0, BIG)], dst_v.at[b, 0], lsem)

    def drain_load(b):
        pltpu.make_async_copy(edge_hbm.at[0, pl.ds(0, BIG)], src_v.at[b, 0], lsem).wait()
        pltpu.make_async_copy(edge_hbm.at[1, pl.ds(0, BIG)], dst_v.at[b, 0], lsem).wait()

    load(0, 0)

    def drain_scatter(b):
        pltpu.make_async_copy(ones_v.at[0], acc_out.at[src_v.at[b, 0]],
                              ssem).wait()
        pltpu.make_async_copy(ones_v.at[0], acc_in.at[dst_v.at[b, 0]],
                              ssem).wait()

    def outer(o, _):
        b = lax.rem(o, 2)
        drain_load(b)

        @pl.when(o > 0)
        def _dr():
            drain_scatter(1 - b)

        @pl.when(o + 1 < OUTER)
        def _next():
            load(o + 1, 1 - b)

        pltpu.async_copy(ones_v.at[0], acc_out.at[src_v.at[b, 0]], ssem,
                         add=True)
        pltpu.async_copy(ones_v.at[0], acc_in.at[dst_v.at[b, 0]], ssem,
                         add=True)
        return _
    lax.fori_loop(0, OUTER, outer, None)
    drain_scatter((OUTER - 1) % 2)

    @pl.when(w < EXTRA_TILES)
    def _extra():
        e0 = estart + BASE_CHUNKS * CH
        pltpu.sync_copy(edge_hbm.at[0, pl.ds(e0, CH)], src_v.at[0, 0, pl.ds(0, CH)])
        pltpu.sync_copy(edge_hbm.at[1, pl.ds(e0, CH)], dst_v.at[0, 0, pl.ds(0, CH)])
        pltpu.sync_copy(ones_v.at[0, pl.ds(0, CH)],
                        acc_out.at[src_v.at[0, 0, pl.ds(0, CH)]], add=True)
        pltpu.sync_copy(ones_v.at[0, pl.ds(0, CH)],
                        acc_in.at[dst_v.at[0, 0, pl.ds(0, CH)]], add=True)

    plsc.subcore_barrier()
    off = s * SLICE

    @pl.when(c == 0)
    def _w0():
        pltpu.sync_copy(acc_out.at[pl.ds(off, SLICE)], outd_a.at[pl.ds(off, SLICE)])
        pltpu.sync_copy(acc_in.at[pl.ds(off, SLICE)], ind_a.at[pl.ds(off, SLICE)])

    @pl.when(c == 1)
    def _w1():
        pltpu.sync_copy(acc_out.at[pl.ds(off, SLICE)], outd_b.at[pl.ds(off, SLICE)])
        pltpu.sync_copy(acc_in.at[pl.ds(off, SLICE)], ind_b.at[pl.ds(off, SLICE)])


def _rsqrt16(x):
    i = lax.bitcast_convert_type(x, jnp.int32)
    i = jnp.int32(0x5F3759DF) - lax.shift_right_logical(i, 1)
    y = lax.bitcast_convert_type(i, jnp.float32)
    for _ in range(3):
        y = y * (1.5 - 0.5 * x * y * y)
    return y


QCH = SLICE // 4


def _sc_edge(edge_hbm, ia_hbm, ib_hbm, t_a, t_b, q_a, q_b,
             src_v, dst_v, val_v, iva_v, ivb_v, qsl_v, q_vmem,
             acc_t, lsem, ssem):
    c = lax.axis_index("c")
    s = lax.axis_index("s")
    w = c * NSUB + s

    def fill_zero(i, _):
        qsl_v[pl.ds(i * 16, 16)] = jnp.zeros((16,), jnp.float32)
        return _
    lax.fori_loop(0, QCH // 16, fill_zero, None)
    for qc in range(4):
        pltpu.sync_copy(qsl_v, acc_t.at[pl.ds(s * SLICE + qc * QCH, QCH)])

    for qc in range(4):
        off = s * SLICE + qc * QCH
        pltpu.sync_copy(ia_hbm.at[pl.ds(off, QCH)], iva_v)
        pltpu.sync_copy(ib_hbm.at[pl.ds(off, QCH)], ivb_v)

        def qbody(i, _):
            d = jnp.maximum(iva_v[pl.ds(i * 16, 16)] + ivb_v[pl.ds(i * 16, 16)],
                            1.0)
            qsl_v[pl.ds(i * 16, 16)] = _rsqrt16(d)
            return _
        lax.fori_loop(0, QCH // 16, qbody, None)

        @pl.when(c == 0)
        def _q0():
            pltpu.sync_copy(qsl_v, q_a.at[pl.ds(off, QCH)])

        @pl.when(c == 1)
        def _q1():
            pltpu.sync_copy(qsl_v, q_b.at[pl.ds(off, QCH)])

    plsc.subcore_barrier()

    @pl.when(c == 0)
    def _l0():
        pltpu.sync_copy(q_a, q_vmem)

    @pl.when(c == 1)
    def _l1():
        pltpu.sync_copy(q_b, q_vmem)

    estart = _tile_chunk_start(w) * CH

    def load(o, b):
        e0 = estart + o * BIG
        pltpu.async_copy(edge_hbm.at[0, pl.ds(e0, BIG)], src_v.at[b, 0], lsem)
        pltpu.async_copy(edge_hbm.at[1, pl.ds(e0, BIG)], dst_v.at[b, 0], lsem)

    def drain_load(b):
        pltpu.make_async_copy(edge_hbm.at[0, pl.ds(0, BIG)], src_v.at[b, 0], lsem).wait()
        pltpu.make_async_copy(edge_hbm.at[1, pl.ds(0, BIG)], dst_v.at[b, 0], lsem).wait()

    load(0, 0)

    def drain_scatter(b):
        pltpu.make_async_copy(val_v.at[b, 0], acc_t.at[src_v.at[b, 0]],
                              ssem).wait()

    def outer(o, _):
        b = lax.rem(o, 2)
        drain_load(b)

        @pl.when(o > 0)
        def _dr():
            drain_scatter(1 - b)

        @pl.when(o + 1 < OUTER)
        def _next():
            load(o + 1, 1 - b)

        for k in range(BIG // 16):
            idx16 = dst_v[b, 0, pl.ds(k * 16, 16)]
            val_v[b, 0, pl.ds(k * 16, 16)] = plsc.load_gather(q_vmem, [idx16])
        pltpu.async_copy(val_v.at[b, 0], acc_t.at[src_v.at[b, 0]], ssem,
                         add=True)
        return _
    lax.fori_loop(0, OUTER, outer, None)
    drain_scatter((OUTER - 1) % 2)

    @pl.when(w < EXTRA_TILES)
    def _extra():
        e0 = estart + BASE_CHUNKS * CH
        pltpu.sync_copy(edge_hbm.at[0, pl.ds(e0, CH)], src_v.at[0, 0, pl.ds(0, CH)])
        pltpu.sync_copy(edge_hbm.at[1, pl.ds(e0, CH)], dst_v.at[0, 0, pl.ds(0, CH)])
        for k in range(CH // 16):
            idx16 = dst_v[0, 0, pl.ds(k * 16, 16)]
            val_v[0, 0, pl.ds(k * 16, 16)] = plsc.load_gather(q_vmem, [idx16])
        pltpu.sync_copy(val_v.at[0, 0, pl.ds(0, CH)],
                        acc_t.at[src_v.at[0, 0, pl.ds(0, CH)]], add=True)

    plsc.subcore_barrier()
    off = s * SLICE

    @pl.when(c == 0)
    def _w0():
        pltpu.sync_copy(acc_t.at[pl.ds(off, SLICE)], t_a.at[pl.ds(off, SLICE)])

    @pl.when(c == 1)
    def _w1():
        pltpu.sync_copy(acc_t.at[pl.ds(off, SLICE)], t_b.at[pl.ds(off, SLICE)])


def _tc_final_body(x_ref, wg_ref, bg_ref, oa_ref, ob_ref, ta_ref, tb_ref,
                   wc_ref, bc_ref, out_ref, acc_ref):
    j = pl.program_id(0)

    @pl.when(j == 0)
    def _init():
        acc_ref[...] = jnp.zeros((8, 128), jnp.float32)

    z = lax.dot_general(wg_ref[...], x_ref[...],
                        (((1,), (1,)), ((), ())),
                        preferred_element_type=jnp.float32)
    z = z + bg_ref[:, 0:1]
    gt = 1.0 / (1.0 + jnp.exp(-z))

    outd = jnp.maximum(oa_ref[...] + ob_ref[...], 1.0)
    cvec = lax.rsqrt(outd) * (ta_ref[...] + tb_ref[...])
    node = j * BN + lax.broadcasted_iota(jnp.int32, (1, BN), 1)
    prod = jnp.where(node < N, gt * cvec, 0.0)
    acc = acc_ref[...]
    for k in range(BN // 128):
        acc = acc + prod[:, k * 128:(k + 1) * 128]
    acc_ref[...] = acc

    @pl.when(j == GRID - 1)
    def _fin():
        s8 = jnp.sum(acc_ref[...], axis=1, keepdims=True)
        out = lax.dot_general(s8, wc_ref[...],
                              (((0,), (0,)), ((), ())),
                              preferred_element_type=jnp.float32)
        out_ref[...] = out + float(N) * bc_ref[...]


@functools.lru_cache(maxsize=None)
def _get_sc_kernels():
    mesh = _get_mesh()
    params = pltpu.CompilerParams(use_tc_tiling_on_sc=False,
                                  needs_layout_passes=False)
    hist = functools.partial(
        pl.kernel,
        mesh=mesh,
        compiler_params=params,
        out_type=[jax.ShapeDtypeStruct((NP,), jnp.float32) for _ in range(4)],
        scratch_types=[
            pltpu.VMEM((2, 1, BIG), jnp.int32),
            pltpu.VMEM((2, 1, BIG), jnp.int32),
            pltpu.VMEM((1, BIG), jnp.float32),
            pltpu.VMEM((SLICE,), jnp.float32),
            pltpu.VMEM_SHARED((NP,), jnp.float32),
            pltpu.VMEM_SHARED((NP,), jnp.float32),
            pltpu.SemaphoreType.DMA,
            pltpu.SemaphoreType.DMA,
        ],
    )(_sc_hist)
    edge = functools.partial(
        pl.kernel,
        mesh=mesh,
        compiler_params=params,
        out_type=[jax.ShapeDtypeStruct((NP,), jnp.float32) for _ in range(4)],
        scratch_types=[
            pltpu.VMEM((2, 1, BIG), jnp.int32),
            pltpu.VMEM((2, 1, BIG), jnp.int32),
            pltpu.VMEM((2, 1, BIG), jnp.float32),
            pltpu.VMEM((QCH,), jnp.float32),
            pltpu.VMEM((QCH,), jnp.float32),
            pltpu.VMEM((QCH,), jnp.float32),
            pltpu.VMEM((NP,), jnp.float32),
            pltpu.VMEM_SHARED((NP,), jnp.float32),
            pltpu.SemaphoreType.DMA,
            pltpu.SemaphoreType.DMA,
        ],
    )(_sc_edge)
    return hist, edge


def kernel(x, edge_index, Wg, bg, Wc, bc):
    sc_hist, sc_edge = _get_sc_kernels()

    outd_a, outd_b, ind_a, ind_b = sc_hist(edge_index)
    t_a, t_b, _, _ = sc_edge(edge_index, ind_a, ind_b)

    Wg8 = jnp.zeros((8, D), jnp.float32).at[:GDIM].set(Wg)
    bg8 = jnp.broadcast_to(
        jnp.zeros((8,), jnp.float32).at[:GDIM].set(bg)[:, None], (8, 128))
    Wc8 = jnp.zeros((8, HDIM), jnp.float32).at[:GDIM].set(Wc)
    out = pl.pallas_call(
        _tc_final_body,
        grid=(GRID,),
        in_specs=[pl.BlockSpec((BN, D), lambda j: (j, 0)),
                  pl.BlockSpec((8, D), lambda j: (0, 0)),
                  pl.BlockSpec((8, 128), lambda j: (0, 0)),
                  pl.BlockSpec((1, BN), lambda j: (0, j)),
                  pl.BlockSpec((1, BN), lambda j: (0, j)),
                  pl.BlockSpec((1, BN), lambda j: (0, j)),
                  pl.BlockSpec((1, BN), lambda j: (0, j)),
                  pl.BlockSpec((8, HDIM), lambda j: (0, 0)),
                  pl.BlockSpec((1, HDIM), lambda j: (0, 0))],
        out_specs=pl.BlockSpec((1, HDIM), lambda j: (0, 0)),
        out_shape=jax.ShapeDtypeStruct((1, HDIM), jnp.float32),
        scratch_shapes=[pltpu.VMEM((8, 128), jnp.float32)],
    )(x, Wg8, bg8,
      outd_a.reshape(1, NP), outd_b.reshape(1, NP),
      t_a.reshape(1, NP), t_b.reshape(1, NP),
      Wc8, bc.reshape(1, HDIM))

    return out
